# Initial kernel scaffold; baseline (speedup 1.0000x reference)
import functools

import jax
import jax.numpy as jnp
from jax import lax
from jax.experimental import pallas as pl
from jax.experimental.pallas import tpu as pltpu
from jax.experimental.pallas import tpu_sc as plsc

N = 10000
E = 320000
F = 128
H1 = 256
H2 = 128
C = 40

NP = 10240
NC, NS = 2, 16
NW = NC * NS
EPW = E // NW
EBLK = 80
NBLK = EPW // EBLK
NODES_PER_TILE = NP // NS

_mesh = plsc.VectorSubcoreMesh(core_axis_name="c", subcore_axis_name="s")
_HIGH = jax.lax.Precision.HIGHEST


def _dot(a, b):
    return jax.lax.dot_general(a, b, (((1,), (0,)), ((), ())),
                               precision=_HIGH,
                               preferred_element_type=jnp.float32)


def _mm1_body(x_ref, wl_ref, wr_ref, xla_ref, xlb_ref, xr_ref):
    xv = x_ref[...]
    xl = _dot(xv, wl_ref[...])
    xla_ref[...] = xl[:, :128]
    xlb_ref[...] = xl[:, 128:]
    xr_ref[...] = _dot(xv, wr_ref[...])


def _tc_mm1(x, Wl1, Wr1):
    blk = 400
    grid = (N // blk,)
    return pl.pallas_call(
        _mm1_body,
        grid=grid,
        in_specs=[
            pl.BlockSpec((blk, F), lambda i: (i, 0)),
            pl.BlockSpec((F, H1), lambda i: (0, 0)),
            pl.BlockSpec((F, H1), lambda i: (0, 0)),
        ],
        out_specs=[
            pl.BlockSpec((blk, 128), lambda i: (i, 0)),
            pl.BlockSpec((blk, 128), lambda i: (i, 0)),
            pl.BlockSpec((blk, H1), lambda i: (i, 0)),
        ],
        out_shape=[
            jax.ShapeDtypeStruct((N, 128), jnp.float32),
            jax.ShapeDtypeStruct((N, 128), jnp.float32),
            jax.ShapeDtypeStruct((N, H1), jnp.float32),
        ],
    )(x, Wl1, Wr1)


def _mid_body(a_ref, b_ref, d_ref, b1_ref, wl2_ref, wr2_ref, xl2_ref, xr2_ref):
    acc_a = a_ref[0] + a_ref[1]
    acc_b = b_ref[0] + b_ref[1]
    rden = 1.0 / (d_ref[0] + d_ref[1] + 1e-16)
    b1v = b1_ref[...]
    ha = jnp.maximum(acc_a * rden + b1v[:, :128], 0.0)
    hb = jnp.maximum(acc_b * rden + b1v[:, 128:], 0.0)
    wl2 = wl2_ref[...]
    wr2 = wr2_ref[...]
    xl2_ref[...] = _dot(ha, wl2[:128, :]) + _dot(hb, wl2[128:, :])
    xr2_ref[...] = _dot(ha, wr2[:128, :]) + _dot(hb, wr2[128:, :])


def _tc_mid(acc_a, acc_b, den_rep, b1r, Wl2, Wr2):
    blk = 512
    grid = (NP // blk,)
    return pl.pallas_call(
        _mid_body,
        grid=grid,
        in_specs=[
            pl.BlockSpec((2, blk, 128), lambda i: (0, i, 0)),
            pl.BlockSpec((2, blk, 128), lambda i: (0, i, 0)),
            pl.BlockSpec((2, blk, 128), lambda i: (0, i, 0)),
            pl.BlockSpec((1, H1), lambda i: (0, 0)),
            pl.BlockSpec((H1, H2), lambda i: (0, 0)),
            pl.BlockSpec((H1, H2), lambda i: (0, 0)),
        ],
        out_specs=[
            pl.BlockSpec((blk, 128), lambda i: (i, 0)),
            pl.BlockSpec((blk, 128), lambda i: (i, 0)),
        ],
        out_shape=[
            jax.ShapeDtypeStruct((NP, 128), jnp.float32),
            jax.ShapeDtypeStruct((NP, 128), jnp.float32),
        ],
    )(acc_a, acc_b, den_rep, b1r, Wl2, Wr2)


def _fin_body(a_ref, d_ref, b2_ref, wc_ref, bc_ref, out_ref):
    acc = a_ref[0] + a_ref[1]
    rden = 1.0 / (d_ref[0] + d_ref[1] + 1e-16)
    h2 = jnp.maximum(acc * rden + b2_ref[...], 0.0)
    out_ref[...] = _dot(h2, wc_ref[...]) + bc_ref[...]


def _tc_final(acc2, den2_rep, b2r, WcP, bcP):
    blk = 512
    grid = (NP // blk,)
    return pl.pallas_call(
        _fin_body,
        grid=grid,
        in_specs=[
            pl.BlockSpec((2, blk, 128), lambda i: (0, i, 0)),
            pl.BlockSpec((2, blk, 128), lambda i: (0, i, 0)),
            pl.BlockSpec((1, 128), lambda i: (0, 0)),
            pl.BlockSpec((128, 128), lambda i: (0, 0)),
            pl.BlockSpec((1, 128), lambda i: (0, 0)),
        ],
        out_specs=pl.BlockSpec((blk, 128), lambda i: (i, 0)),
        out_shape=jax.ShapeDtypeStruct((NP, 128), jnp.float32),
    )(acc2, den2_rep, b2r, WcP, bcP)


def _zero_vmem_2d(ref, rows):
    @pl.loop(0, rows)
    def _(r):
        for c in range(ref.shape[1] // 16):
            ref[r, pl.ds(16 * c, 16)] = jnp.zeros((16,), jnp.float32)


def _zero_vmem_1d(ref, n):
    @pl.loop(0, n // 16)
    def _(i):
        ref[pl.ds(16 * i, 16)] = jnp.zeros((16,), jnp.float32)


def _edge_scores(nch, left_of, right_of, att_chunks, scr_v, ex_v):
    @pl.loop(0, EBLK // 16)
    def _(g):
        @pl.loop(0, 16)
        def _(e2):
            ee = 16 * g + e2
            acc = jnp.zeros((16,), jnp.float32)
            for c in range(nch):
                lref, lc = left_of(c)
                lv = lref[ee, pl.ds(16 * lc, 16)]
                rref, rc = right_of(c)
                rv = rref[ee, pl.ds(16 * rc, 16)]
                m = lv + rv
                m = jnp.maximum(m, 0.2 * m)
                acc = acc + att_chunks[c] * m
            scr_v[pl.ds(16 * e2, 16)] = acc

        ev = jnp.zeros((16,), jnp.float32)
        lanes = jnp.arange(16, dtype=jnp.int32) * 16
        for c in range(16):
            ev = ev + plsc.load_gather(scr_v, [lanes + c])
        ex_v[pl.ds(16 * g, 16)] = jnp.exp(ev)


def _scale_rows(buf, ex_v):
    @pl.loop(0, EBLK)
    def _(ee):
        idx = jnp.broadcast_to(ee, (16,)).astype(jnp.int32)
        exb = plsc.load_gather(ex_v, [idx])
        for c in range(buf.shape[1] // 16):
            buf[ee, pl.ds(16 * c, 16)] = buf[ee, pl.ds(16 * c, 16)] * exb


def _write_stripe(sp_ref, out_ref, core, s, zbuf):
    @pl.loop(0, NODES_PER_TILE // 128)
    def _(i):
        r0 = NODES_PER_TILE * s + 128 * i
        pltpu.sync_copy(sp_ref.at[pl.ds(r0, 128)], zbuf)
        pltpu.sync_copy(zbuf, out_ref.at[core, pl.ds(r0, 128)])


def _write_den_stripe(den_sp, out_ref, core, s, zden, zbuf):
    pltpu.sync_copy(den_sp.at[pl.ds(NODES_PER_TILE * s, NODES_PER_TILE)], zden)

    @pl.loop(0, NODES_PER_TILE // 128)
    def _(i):
        @pl.loop(0, 128)
        def _(r):
            idx = jnp.broadcast_to(128 * i + r, (16,)).astype(jnp.int32)
            dv = plsc.load_gather(zden, [idx])
            for c in range(8):
                zbuf[r, pl.ds(16 * c, 16)] = dv
        pltpu.sync_copy(zbuf, out_ref.at[core, pl.ds(NODES_PER_TILE * s + 128 * i, 128)])


def _sc_pass1_l1(xla, xlb, xr, src, dst, att):
    @functools.partial(
        pl.kernel,
        mesh=_mesh,
        out_type=[
            jax.ShapeDtypeStruct((E,), jnp.float32),
            jax.ShapeDtypeStruct((NC, NP, 128), jnp.float32),
            jax.ShapeDtypeStruct((NC, NP, 128), jnp.float32),
        ],
        scratch_types=[
            pltpu.VMEM((EBLK,), jnp.int32),
            pltpu.VMEM((EBLK,), jnp.int32),
            pltpu.VMEM((EBLK, 128), jnp.float32),
            pltpu.VMEM((EBLK, 128), jnp.float32),
            pltpu.VMEM((EBLK, H1), jnp.float32),
            pltpu.VMEM((EBLK,), jnp.float32),
            pltpu.VMEM((256,), jnp.float32),
            pltpu.VMEM((H1,), jnp.float32),
            pltpu.VMEM((128, 128), jnp.float32),
            pltpu.VMEM((NODES_PER_TILE,), jnp.float32),
            pltpu.VMEM_SHARED((NP, 128), jnp.float32),
            pltpu.VMEM_SHARED((NP,), jnp.float32),
        ],
    )
    def k(xla_h, xlb_h, xr_h, src_h, dst_h, att_h,
          ex_h, acc_h, den_h,
          isrc, idst, bufA, bufB, bufR, ex_v, scr_v, att_v, zbuf, zden,
          acc_sp, den_sp):
        core = lax.axis_index("c")
        s = lax.axis_index("s")
        w = core * NS + s

        pltpu.sync_copy(att_h, att_v)
        att_chunks = [att_v[pl.ds(16 * c, 16)] for c in range(H1 // 16)]

        _zero_vmem_2d(zbuf, 128)
        _zero_vmem_1d(zden, NODES_PER_TILE)

        @pl.loop(0, NODES_PER_TILE // 128)
        def _(i):
            pltpu.sync_copy(zbuf, acc_sp.at[pl.ds(NODES_PER_TILE * s + 128 * i, 128)])
        pltpu.sync_copy(zden, den_sp.at[pl.ds(NODES_PER_TILE * s, NODES_PER_TILE)])
        plsc.subcore_barrier()

        def left_of(c):
            return (bufA, c) if c < 8 else (bufB, c - 8)

        def right_of(c):
            return (bufR, c)

        @pl.loop(0, NBLK)
        def _(b):
            base = w * EPW + EBLK * b
            pltpu.sync_copy(src_h.at[pl.ds(base, EBLK)], isrc)
            pltpu.sync_copy(dst_h.at[pl.ds(base, EBLK)], idst)
            pltpu.sync_copy(xla_h.at[isrc], bufA)
            pltpu.sync_copy(xlb_h.at[isrc], bufB)
            pltpu.sync_copy(xr_h.at[idst], bufR)

            _edge_scores(H1 // 16, left_of, right_of, att_chunks, scr_v, ex_v)

            pltpu.sync_copy(ex_v, ex_h.at[pl.ds(base, EBLK)])
            pltpu.sync_copy(ex_v, den_sp.at[idst], add=True)
            _scale_rows(bufA, ex_v)
            pltpu.sync_copy(bufA, acc_sp.at[idst], add=True)

        plsc.subcore_barrier()
        _write_stripe(acc_sp, acc_h, core, s, zbuf)
        _write_den_stripe(den_sp, den_h, core, s, zden, zbuf)

    return k(xla, xlb, xr, src, dst, att)


def _sc_pass2_l1(xlb, ex, src, dst):
    @functools.partial(
        pl.kernel,
        mesh=_mesh,
        out_type=jax.ShapeDtypeStruct((NC, NP, 128), jnp.float32),
        scratch_types=[
            pltpu.VMEM((EBLK,), jnp.int32),
            pltpu.VMEM((EBLK,), jnp.int32),
            pltpu.VMEM((EBLK, 128), jnp.float32),
            pltpu.VMEM((EBLK,), jnp.float32),
            pltpu.VMEM((128, 128), jnp.float32),
            pltpu.VMEM_SHARED((NP, 128), jnp.float32),
        ],
    )
    def k(xlb_h, ex_h, src_h, dst_h, acc_h,
          isrc, idst, bufA, ex_v, zbuf, acc_sp):
        core = lax.axis_index("c")
        s = lax.axis_index("s")
        w = core * NS + s

        _zero_vmem_2d(zbuf, 128)

        @pl.loop(0, NODES_PER_TILE // 128)
        def _(i):
            pltpu.sync_copy(zbuf, acc_sp.at[pl.ds(NODES_PER_TILE * s + 128 * i, 128)])
        plsc.subcore_barrier()

        @pl.loop(0, NBLK)
        def _(b):
            base = w * EPW + EBLK * b
            pltpu.sync_copy(src_h.at[pl.ds(base, EBLK)], isrc)
            pltpu.sync_copy(dst_h.at[pl.ds(base, EBLK)], idst)
            pltpu.sync_copy(xlb_h.at[isrc], bufA)
            pltpu.sync_copy(ex_h.at[pl.ds(base, EBLK)], ex_v)
            _scale_rows(bufA, ex_v)
            pltpu.sync_copy(bufA, acc_sp.at[idst], add=True)

        plsc.subcore_barrier()
        _write_stripe(acc_sp, acc_h, core, s, zbuf)

    return k(xlb, ex, src, dst)


def _sc_pass_l2(xl2, xr2, src, dst, att):
    @functools.partial(
        pl.kernel,
        mesh=_mesh,
        out_type=[
            jax.ShapeDtypeStruct((NC, NP, 128), jnp.float32),
            jax.ShapeDtypeStruct((NC, NP, 128), jnp.float32),
        ],
        scratch_types=[
            pltpu.VMEM((EBLK,), jnp.int32),
            pltpu.VMEM((EBLK,), jnp.int32),
            pltpu.VMEM((EBLK, 128), jnp.float32),
            pltpu.VMEM((EBLK, 128), jnp.float32),
            pltpu.VMEM((EBLK,), jnp.float32),
            pltpu.VMEM((256,), jnp.float32),
            pltpu.VMEM((H2,), jnp.float32),
            pltpu.VMEM((128, 128), jnp.float32),
            pltpu.VMEM((NODES_PER_TILE,), jnp.float32),
            pltpu.VMEM_SHARED((NP, 128), jnp.float32),
            pltpu.VMEM_SHARED((NP,), jnp.float32),
        ],
    )
    def k(xl2_h, xr2_h, src_h, dst_h, att_h,
          acc_h, den_h,
          isrc, idst, bufA, bufR, ex_v, scr_v, att_v, zbuf, zden,
          acc_sp, den_sp):
        core = lax.axis_index("c")
        s = lax.axis_index("s")
        w = core * NS + s

        pltpu.sync_copy(att_h, att_v)
        att_chunks = [att_v[pl.ds(16 * c, 16)] for c in range(H2 // 16)]

        _zero_vmem_2d(zbuf, 128)
        _zero_vmem_1d(zden, NODES_PER_TILE)

        @pl.loop(0, NODES_PER_TILE // 128)
        def _(i):
            pltpu.sync_copy(zbuf, acc_sp.at[pl.ds(NODES_PER_TILE * s + 128 * i, 128)])
        pltpu.sync_copy(zden, den_sp.at[pl.ds(NODES_PER_TILE * s, NODES_PER_TILE)])
        plsc.subcore_barrier()

        def left_of(c):
            return (bufA, c)

        def right_of(c):
            return (bufR, c)

        @pl.loop(0, NBLK)
        def _(b):
            base = w * EPW + EBLK * b
            pltpu.sync_copy(src_h.at[pl.ds(base, EBLK)], isrc)
            pltpu.sync_copy(dst_h.at[pl.ds(base, EBLK)], idst)
            pltpu.sync_copy(xl2_h.at[isrc], bufA)
            pltpu.sync_copy(xr2_h.at[idst], bufR)

            _edge_scores(H2 // 16, left_of, right_of, att_chunks, scr_v, ex_v)

            pltpu.sync_copy(ex_v, den_sp.at[idst], add=True)
            _scale_rows(bufA, ex_v)
            pltpu.sync_copy(bufA, acc_sp.at[idst], add=True)

        plsc.subcore_barrier()
        _write_stripe(acc_sp, acc_h, core, s, zbuf)
        _write_den_stripe(den_sp, den_h, core, s, zden, zbuf)

    return k(xl2, xr2, src, dst, att)


def kernel(x, edge_index, Wl1, Wr1, att1, b1, Wl2, Wr2, att2, b2, Wc, bc):
    src = edge_index[0].astype(jnp.int32)
    dst = edge_index[1].astype(jnp.int32)

    xla, xlb, xr1 = _tc_mm1(x, Wl1, Wr1)

    ex1, acc1a, den1_rep = _sc_pass1_l1(xla, xlb, xr1, src, dst, att1)
    acc1b = _sc_pass2_l1(xlb, ex1, src, dst)

    xl2, xr2 = _tc_mid(acc1a, acc1b, den1_rep,
                       b1.reshape(1, H1), Wl2, Wr2)

    acc2, den2_rep = _sc_pass_l2(xl2, xr2, src, dst, att2)

    WcP = jnp.pad(Wc, ((0, 0), (0, 128 - C)))
    bcP = jnp.pad(bc, (0, 128 - C)).reshape(1, 128)
    outP = _tc_final(acc2, den2_rep, b2.reshape(1, H2), WcP, bcP)
    return outP[:N, :C]


# speedup vs baseline: 5.9700x; 5.9700x over previous
import dataclasses
import functools

import jax
import jax.numpy as jnp
from jax import lax
from jax.experimental import pallas as pl
from jax.experimental.pallas import tpu as pltpu
from jax.experimental.pallas import tpu_sc as plsc

N = 10000
E = 320000
F = 128
H1 = 256
H2 = 128
C = 40

NP = 10240
NC, NS = 2, 16
NW = NC * NS
EPW = E // NW
EBLK = 80
NBLK = EPW // EBLK
NODES_PER_TILE = NP // NS

_mesh = plsc.VectorSubcoreMesh(core_axis_name="c", subcore_axis_name="s")
_SC_PARAMS = pltpu.CompilerParams()
if "needs_layout_passes" in pltpu.CompilerParams.__dataclass_fields__:
    _SC_PARAMS = dataclasses.replace(_SC_PARAMS, needs_layout_passes=False)
_HIGH = jax.lax.Precision.HIGHEST


def _dot(a, b):
    return jax.lax.dot_general(a, b, (((1,), (0,)), ((), ())),
                               precision=_HIGH,
                               preferred_element_type=jnp.float32)


def _mm1_body(x_ref, wl_ref, wr_ref, xla_ref, xlb_ref, xr_ref):
    xv = x_ref[...]
    xl = _dot(xv, wl_ref[...])
    xla_ref[...] = xl[:, :128]
    xlb_ref[...] = xl[:, 128:]
    xr_ref[...] = _dot(xv, wr_ref[...])


def _tc_mm1(x, Wl1, Wr1):
    blk = 400
    grid = (N // blk,)
    return pl.pallas_call(
        _mm1_body,
        grid=grid,
        in_specs=[
            pl.BlockSpec((blk, F), lambda i: (i, 0)),
            pl.BlockSpec((F, H1), lambda i: (0, 0)),
            pl.BlockSpec((F, H1), lambda i: (0, 0)),
        ],
        out_specs=[
            pl.BlockSpec((blk, 128), lambda i: (i, 0)),
            pl.BlockSpec((blk, 128), lambda i: (i, 0)),
            pl.BlockSpec((blk, H1), lambda i: (i, 0)),
        ],
        out_shape=[
            jax.ShapeDtypeStruct((N, 128), jnp.float32),
            jax.ShapeDtypeStruct((N, 128), jnp.float32),
            jax.ShapeDtypeStruct((N, H1), jnp.float32),
        ],
    )(x, Wl1, Wr1)


def _mid_body(a_ref, b_ref, d_ref, b1_ref, wl2_ref, wr2_ref, xl2_ref, xr2_ref):
    acc_a = a_ref[0] + a_ref[1]
    acc_b = b_ref[0] + b_ref[1]
    rden = 1.0 / (d_ref[0] + d_ref[1] + 1e-16)
    b1v = b1_ref[...]
    ha = jnp.maximum(acc_a * rden + b1v[:, :128], 0.0)
    hb = jnp.maximum(acc_b * rden + b1v[:, 128:], 0.0)
    wl2 = wl2_ref[...]
    wr2 = wr2_ref[...]
    xl2_ref[...] = _dot(ha, wl2[:128, :]) + _dot(hb, wl2[128:, :])
    xr2_ref[...] = _dot(ha, wr2[:128, :]) + _dot(hb, wr2[128:, :])


def _tc_mid(acc_a, acc_b, den_rep, b1r, Wl2, Wr2):
    blk = 512
    grid = (NP // blk,)
    return pl.pallas_call(
        _mid_body,
        grid=grid,
        in_specs=[
            pl.BlockSpec((2, blk, 128), lambda i: (0, i, 0)),
            pl.BlockSpec((2, blk, 128), lambda i: (0, i, 0)),
            pl.BlockSpec((2, blk, 128), lambda i: (0, i, 0)),
            pl.BlockSpec((1, H1), lambda i: (0, 0)),
            pl.BlockSpec((H1, H2), lambda i: (0, 0)),
            pl.BlockSpec((H1, H2), lambda i: (0, 0)),
        ],
        out_specs=[
            pl.BlockSpec((blk, 128), lambda i: (i, 0)),
            pl.BlockSpec((blk, 128), lambda i: (i, 0)),
        ],
        out_shape=[
            jax.ShapeDtypeStruct((NP, 128), jnp.float32),
            jax.ShapeDtypeStruct((NP, 128), jnp.float32),
        ],
    )(acc_a, acc_b, den_rep, b1r, Wl2, Wr2)


def _fin_body(a_ref, d_ref, b2_ref, wc_ref, bc_ref, out_ref):
    acc = a_ref[0] + a_ref[1]
    rden = 1.0 / (d_ref[0] + d_ref[1] + 1e-16)
    h2 = jnp.maximum(acc * rden + b2_ref[...], 0.0)
    out_ref[...] = _dot(h2, wc_ref[...]) + bc_ref[...]


def _tc_final(acc2, den2_rep, b2r, WcP, bcP):
    blk = 512
    grid = (NP // blk,)
    return pl.pallas_call(
        _fin_body,
        grid=grid,
        in_specs=[
            pl.BlockSpec((2, blk, 128), lambda i: (0, i, 0)),
            pl.BlockSpec((2, blk, 128), lambda i: (0, i, 0)),
            pl.BlockSpec((1, 128), lambda i: (0, 0)),
            pl.BlockSpec((128, 128), lambda i: (0, 0)),
            pl.BlockSpec((1, 128), lambda i: (0, 0)),
        ],
        out_specs=pl.BlockSpec((blk, 128), lambda i: (i, 0)),
        out_shape=jax.ShapeDtypeStruct((NP, 128), jnp.float32),
    )(acc2, den2_rep, b2r, WcP, bcP)


def _zero_vmem_2d(ref, rows):
    @pl.loop(0, rows)
    def _(r):
        for c in range(ref.shape[1] // 16):
            ref[r, pl.ds(16 * c, 16)] = jnp.zeros((16,), jnp.float32)


def _zero_vmem_1d(ref, n):
    @pl.loop(0, n // 16)
    def _(i):
        ref[pl.ds(16 * i, 16)] = jnp.zeros((16,), jnp.float32)


def _edge_scores(nch, left_of, right_of, att_chunks, scr_v, ex_v):
    @pl.loop(0, EBLK // 16)
    def _(g):
        @pl.loop(0, 16)
        def _(e2):
            ee = 16 * g + e2
            acc = jnp.zeros((16,), jnp.float32)
            for c in range(nch):
                lref, lc = left_of(c)
                lv = lref[ee, pl.ds(16 * lc, 16)]
                rref, rc = right_of(c)
                rv = rref[ee, pl.ds(16 * rc, 16)]
                m = lv + rv
                m = jnp.maximum(m, 0.2 * m)
                acc = acc + att_chunks[c] * m
            scr_v[pl.ds(16 * e2, 16)] = acc

        ev = jnp.zeros((16,), jnp.float32)
        lanes = jnp.arange(16, dtype=jnp.int32) * 16
        for c in range(16):
            ev = ev + plsc.load_gather(scr_v, [lanes + c])
        ex_v[pl.ds(16 * g, 16)] = jnp.exp(ev)


def _scale_rows(buf, ex_v):
    @pl.loop(0, EBLK)
    def _(ee):
        idx = jnp.broadcast_to(ee, (16,)).astype(jnp.int32)
        exb = plsc.load_gather(ex_v, [idx])
        for c in range(buf.shape[1] // 16):
            buf[ee, pl.ds(16 * c, 16)] = buf[ee, pl.ds(16 * c, 16)] * exb


def _write_stripe(sp_ref, out_ref, core, s, bounce):
    @pl.loop(0, NODES_PER_TILE // 80)
    def _(i):
        r0 = NODES_PER_TILE * s + 80 * i
        pltpu.sync_copy(sp_ref.at[pl.ds(r0, 80)], bounce)
        pltpu.sync_copy(bounce, out_ref.at[core, pl.ds(r0, 80)])


def _write_den_stripe(den_sp, out_ref, core, s, zden, bounce):
    pltpu.sync_copy(den_sp.at[pl.ds(NODES_PER_TILE * s, NODES_PER_TILE)], zden)

    @pl.loop(0, NODES_PER_TILE // 80)
    def _(i):
        @pl.loop(0, 80)
        def _(r):
            idx = jnp.broadcast_to(80 * i + r, (16,)).astype(jnp.int32)
            dv = plsc.load_gather(zden, [idx])
            for c in range(8):
                bounce[r, pl.ds(16 * c, 16)] = dv
        pltpu.sync_copy(bounce, out_ref.at[core, pl.ds(NODES_PER_TILE * s + 80 * i, 80)])


def _sc_pass1_l1(xla, xlb, xr, src, dst, att):
    @functools.partial(
        pl.kernel,
        mesh=_mesh,
        compiler_params=_SC_PARAMS,
        out_type=[
            jax.ShapeDtypeStruct((E,), jnp.float32),
            jax.ShapeDtypeStruct((NC, NP, 128), jnp.float32),
            jax.ShapeDtypeStruct((NC, NP, 128), jnp.float32),
        ],
        scratch_types=[
            pltpu.VMEM((EBLK,), jnp.int32),
            pltpu.VMEM((EBLK,), jnp.int32),
            pltpu.VMEM((EBLK, 128), jnp.float32),
            pltpu.VMEM((EBLK, 128), jnp.float32),
            pltpu.VMEM((EBLK, H1), jnp.float32),
            pltpu.VMEM((EBLK,), jnp.float32),
            pltpu.VMEM((256,), jnp.float32),
            pltpu.VMEM((H1,), jnp.float32),
            pltpu.VMEM((NODES_PER_TILE,), jnp.float32),
            pltpu.VMEM_SHARED((NP, 128), jnp.float32),
            pltpu.VMEM_SHARED((NP,), jnp.float32),
        ],
    )
    def k(xla_h, xlb_h, xr_h, src_h, dst_h, att_h,
          ex_h, acc_h, den_h,
          isrc, idst, bufA, bufB, bufR, ex_v, scr_v, att_v, zden,
          acc_sp, den_sp):
        core = lax.axis_index("c")
        s = lax.axis_index("s")
        w = core * NS + s

        pltpu.sync_copy(att_h, att_v)
        att_chunks = [att_v[pl.ds(16 * c, 16)] for c in range(H1 // 16)]

        _zero_vmem_2d(bufA, EBLK)
        _zero_vmem_1d(zden, NODES_PER_TILE)

        @pl.loop(0, NODES_PER_TILE // 80)
        def _(i):
            pltpu.sync_copy(bufA, acc_sp.at[pl.ds(NODES_PER_TILE * s + 80 * i, 80)])
        pltpu.sync_copy(zden, den_sp.at[pl.ds(NODES_PER_TILE * s, NODES_PER_TILE)])
        plsc.subcore_barrier()

        def left_of(c):
            return (bufA, c) if c < 8 else (bufB, c - 8)

        def right_of(c):
            return (bufR, c)

        @pl.loop(0, NBLK)
        def _(b):
            base = w * EPW + EBLK * b
            pltpu.sync_copy(src_h.at[pl.ds(base, EBLK)], isrc)
            pltpu.sync_copy(dst_h.at[pl.ds(base, EBLK)], idst)
            pltpu.sync_copy(xla_h.at[isrc], bufA)
            pltpu.sync_copy(xlb_h.at[isrc], bufB)
            pltpu.sync_copy(xr_h.at[idst], bufR)

            _edge_scores(H1 // 16, left_of, right_of, att_chunks, scr_v, ex_v)

            pltpu.sync_copy(ex_v, ex_h.at[pl.ds(base, EBLK)])
            pltpu.sync_copy(ex_v, den_sp.at[idst], add=True)
            _scale_rows(bufA, ex_v)
            pltpu.sync_copy(bufA, acc_sp.at[idst], add=True)

        plsc.subcore_barrier()
        _write_stripe(acc_sp, acc_h, core, s, bufA)
        _write_den_stripe(den_sp, den_h, core, s, zden, bufA)

    return k(xla, xlb, xr, src, dst, att)


def _sc_pass2_l1(xlb, ex, src, dst):
    @functools.partial(
        pl.kernel,
        mesh=_mesh,
        compiler_params=_SC_PARAMS,
        out_type=jax.ShapeDtypeStruct((NC, NP, 128), jnp.float32),
        scratch_types=[
            pltpu.VMEM((EBLK,), jnp.int32),
            pltpu.VMEM((EBLK,), jnp.int32),
            pltpu.VMEM((EBLK, 128), jnp.float32),
            pltpu.VMEM((EBLK,), jnp.float32),
            pltpu.VMEM_SHARED((NP, 128), jnp.float32),
        ],
    )
    def k(xlb_h, ex_h, src_h, dst_h, acc_h,
          isrc, idst, bufA, ex_v, acc_sp):
        core = lax.axis_index("c")
        s = lax.axis_index("s")
        w = core * NS + s

        _zero_vmem_2d(bufA, EBLK)

        @pl.loop(0, NODES_PER_TILE // 80)
        def _(i):
            pltpu.sync_copy(bufA, acc_sp.at[pl.ds(NODES_PER_TILE * s + 80 * i, 80)])
        plsc.subcore_barrier()

        @pl.loop(0, NBLK)
        def _(b):
            base = w * EPW + EBLK * b
            pltpu.sync_copy(src_h.at[pl.ds(base, EBLK)], isrc)
            pltpu.sync_copy(dst_h.at[pl.ds(base, EBLK)], idst)
            pltpu.sync_copy(xlb_h.at[isrc], bufA)
            pltpu.sync_copy(ex_h.at[pl.ds(base, EBLK)], ex_v)
            _scale_rows(bufA, ex_v)
            pltpu.sync_copy(bufA, acc_sp.at[idst], add=True)

        plsc.subcore_barrier()
        _write_stripe(acc_sp, acc_h, core, s, bufA)

    return k(xlb, ex, src, dst)


def _sc_pass_l2(xl2, xr2, src, dst, att):
    @functools.partial(
        pl.kernel,
        mesh=_mesh,
        compiler_params=_SC_PARAMS,
        out_type=[
            jax.ShapeDtypeStruct((NC, NP, 128), jnp.float32),
            jax.ShapeDtypeStruct((NC, NP, 128), jnp.float32),
        ],
        scratch_types=[
            pltpu.VMEM((EBLK,), jnp.int32),
            pltpu.VMEM((EBLK,), jnp.int32),
            pltpu.VMEM((EBLK, 128), jnp.float32),
            pltpu.VMEM((EBLK, 128), jnp.float32),
            pltpu.VMEM((EBLK,), jnp.float32),
            pltpu.VMEM((256,), jnp.float32),
            pltpu.VMEM((H2,), jnp.float32),
            pltpu.VMEM((NODES_PER_TILE,), jnp.float32),
            pltpu.VMEM_SHARED((NP, 128), jnp.float32),
            pltpu.VMEM_SHARED((NP,), jnp.float32),
        ],
    )
    def k(xl2_h, xr2_h, src_h, dst_h, att_h,
          acc_h, den_h,
          isrc, idst, bufA, bufR, ex_v, scr_v, att_v, zden,
          acc_sp, den_sp):
        core = lax.axis_index("c")
        s = lax.axis_index("s")
        w = core * NS + s

        pltpu.sync_copy(att_h, att_v)
        att_chunks = [att_v[pl.ds(16 * c, 16)] for c in range(H2 // 16)]

        _zero_vmem_2d(bufA, EBLK)
        _zero_vmem_1d(zden, NODES_PER_TILE)

        @pl.loop(0, NODES_PER_TILE // 80)
        def _(i):
            pltpu.sync_copy(bufA, acc_sp.at[pl.ds(NODES_PER_TILE * s + 80 * i, 80)])
        pltpu.sync_copy(zden, den_sp.at[pl.ds(NODES_PER_TILE * s, NODES_PER_TILE)])
        plsc.subcore_barrier()

        def left_of(c):
            return (bufA, c)

        def right_of(c):
            return (bufR, c)

        @pl.loop(0, NBLK)
        def _(b):
            base = w * EPW + EBLK * b
            pltpu.sync_copy(src_h.at[pl.ds(base, EBLK)], isrc)
            pltpu.sync_copy(dst_h.at[pl.ds(base, EBLK)], idst)
            pltpu.sync_copy(xl2_h.at[isrc], bufA)
            pltpu.sync_copy(xr2_h.at[idst], bufR)

            _edge_scores(H2 // 16, left_of, right_of, att_chunks, scr_v, ex_v)

            pltpu.sync_copy(ex_v, den_sp.at[idst], add=True)
            _scale_rows(bufA, ex_v)
            pltpu.sync_copy(bufA, acc_sp.at[idst], add=True)

        plsc.subcore_barrier()
        _write_stripe(acc_sp, acc_h, core, s, bufA)
        _write_den_stripe(den_sp, den_h, core, s, zden, bufA)

    return k(xl2, xr2, src, dst, att)


def kernel(x, edge_index, Wl1, Wr1, att1, b1, Wl2, Wr2, att2, b2, Wc, bc):
    src = edge_index[0].astype(jnp.int32)
    dst = edge_index[1].astype(jnp.int32)

    xla, xlb, xr1 = _tc_mm1(x, Wl1, Wr1)

    ex1, acc1a, den1_rep = _sc_pass1_l1(xla, xlb, xr1, src, dst, att1)
    acc1b = _sc_pass2_l1(xlb, ex1, src, dst)

    xl2, xr2 = _tc_mid(acc1a, acc1b, den1_rep,
                       b1.reshape(1, H1), Wl2, Wr2)

    acc2, den2_rep = _sc_pass_l2(xl2, xr2, src, dst, att2)

    WcP = jnp.pad(Wc, ((0, 0), (0, 128 - C)))
    bcP = jnp.pad(bc, (0, 128 - C)).reshape(1, 128)
    outP = _tc_final(acc2, den2_rep, b2.reshape(1, H2), WcP, bcP)
    return outP[:N, :C]
